# Initial kernel scaffold; baseline (speedup 1.0000x reference)
import jax
import jax.numpy as jnp
from jax.experimental import pallas as pl


def kernel(H, edge_index, adj_values, gamma, beta, moving_mean, moving_var, W, b):
    raise NotImplementedError("write your pallas kernel here")


# speedup vs baseline: 2.8427x; 2.8427x over previous
import functools

import jax
import jax.numpy as jnp
from jax import lax
from jax.experimental import pallas as pl
from jax.experimental.pallas import tpu as pltpu
from jax.experimental.pallas import tpu_sc as plsc

_N_NODES = 10000
_N_EDGES = 320000
_D = 128
_BN_EPS = 1e-3

_NC = 2
_NS = 16
_NTILES = _NC * _NS
_E_PER_TILE = _N_EDGES // _NTILES
_E_CHK = 80
_N_CHK = _E_PER_TILE // _E_CHK
_ROWS_PER_TILE = 624
_ROWS_REM = _N_NODES - _NS * _ROWS_PER_TILE
_ZBUF = 208


def _sc_body(h_hbm, row_hbm, col_hbm, val_hbm, out_hbm,
             row_v, col_v, val_v, rows_v, zeros_v, acc_sh, sem):
    c = lax.axis_index("c")
    s = lax.axis_index("s")
    tile = c * _NS + s

    def _zrow(i, _):
        for j in range(_D // 16):
            zeros_v[i, pl.ds(j * 16, 16)] = jnp.zeros((16,), jnp.float32)
        return 0
    lax.fori_loop(0, _ZBUF, _zrow, 0)
    for k in range(_ROWS_PER_TILE // _ZBUF):
        pltpu.sync_copy(zeros_v, acc_sh.at[pl.ds(s * _ROWS_PER_TILE + k * _ZBUF, _ZBUF)])

    @pl.when(s == 0)
    def _zero_rem():
        pltpu.sync_copy(zeros_v.at[pl.ds(0, _ROWS_REM)],
                        acc_sh.at[pl.ds(_NS * _ROWS_PER_TILE, _ROWS_REM)])
    plsc.subcore_barrier()

    e_base = tile * _E_PER_TILE

    def _chunk(g, _):
        base = e_base + g * _E_CHK
        pltpu.sync_copy(row_hbm.at[pl.ds(base, _E_CHK)], row_v)
        pltpu.sync_copy(col_hbm.at[pl.ds(base, _E_CHK)], col_v)
        pltpu.sync_copy(val_hbm.at[pl.ds(base, _E_CHK)], val_v)
        pltpu.async_copy(h_hbm.at[col_v], rows_v, sem).wait()

        def _edge(e, _):
            vb = val_v[e]
            for j in range(_D // 16):
                sl = pl.ds(j * 16, 16)
                rows_v[e, sl] = rows_v[e, sl] * vb
            return 0
        lax.fori_loop(0, _E_CHK, _edge, 0)

        pltpu.sync_copy(rows_v, acc_sh.at[row_v], add=True)
        return 0
    lax.fori_loop(0, _N_CHK, _chunk, 0)

    plsc.subcore_barrier()

    pltpu.sync_copy(acc_sh.at[pl.ds(s * _ROWS_PER_TILE, _ROWS_PER_TILE)],
                    out_hbm.at[c, pl.ds(s * _ROWS_PER_TILE, _ROWS_PER_TILE)])

    @pl.when(s == 0)
    def _write_rem():
        pltpu.sync_copy(acc_sh.at[pl.ds(_NS * _ROWS_PER_TILE, _ROWS_REM)],
                        out_hbm.at[c, pl.ds(_NS * _ROWS_PER_TILE, _ROWS_REM)])


@jax.jit
def _sc_segment_sum(H, row, col, val):
    mesh = plsc.VectorSubcoreMesh(core_axis_name="c", subcore_axis_name="s")
    return pl.kernel(
        _sc_body,
        out_type=jax.ShapeDtypeStruct((_NC, _N_NODES, _D), jnp.float32),
        mesh=mesh,
        scratch_types=[
            pltpu.VMEM((_E_CHK,), jnp.int32),
            pltpu.VMEM((_E_CHK,), jnp.int32),
            pltpu.VMEM((_E_CHK, 16), jnp.float32),
            pltpu.VMEM((_E_CHK, _D), jnp.float32),
            pltpu.VMEM((_ZBUF, _D), jnp.float32),
            pltpu.VMEM_SHARED((_N_NODES, _D), jnp.float32),
            pltpu.SemaphoreType.DMA,
        ],
    )(H, row, col, val)


def _tc_body(h0_ref, h1_ref, scale_ref, shift_ref, w_ref, b_ref, o_ref):
    x = h0_ref[...] + h1_ref[...]
    x = x * scale_ref[...] + shift_ref[...]
    y = jnp.dot(x, w_ref[...], preferred_element_type=jnp.float32) + b_ref[...]
    o_ref[...] = 0.5 * y * (1.0 + lax.erf(y * 0.7071067811865476))


@jax.jit
def _tc_ffn(h0, h1, scale, shift, W, b):
    blk = 1000
    grid = (_N_NODES // blk,)
    return pl.pallas_call(
        _tc_body,
        grid=grid,
        in_specs=[
            pl.BlockSpec((blk, _D), lambda i: (i, 0)),
            pl.BlockSpec((blk, _D), lambda i: (i, 0)),
            pl.BlockSpec((1, _D), lambda i: (0, 0)),
            pl.BlockSpec((1, _D), lambda i: (0, 0)),
            pl.BlockSpec((_D, _D), lambda i: (0, 0)),
            pl.BlockSpec((1, _D), lambda i: (0, 0)),
        ],
        out_specs=pl.BlockSpec((blk, _D), lambda i: (i, 0)),
        out_shape=jax.ShapeDtypeStruct((_N_NODES, _D), jnp.float32),
    )(h0, h1, scale, shift, W, b)


def kernel(H, edge_index, adj_values, gamma, beta, moving_mean, moving_var, W, b):
    row = edge_index[0]
    col = edge_index[1]
    val_b = jnp.broadcast_to(adj_values[:, None], (_N_EDGES, 16))
    hpart = _sc_segment_sum(H, row, col, val_b)
    scale = gamma * lax.rsqrt(moving_var + _BN_EPS)
    shift = beta - moving_mean * scale
    return _tc_ffn(hpart[0], hpart[1], scale.reshape(1, _D),
                   shift.reshape(1, _D), W, b.reshape(1, _D))
